# baseline (speedup 1.0000x reference)
import functools

import jax
import jax.numpy as jnp
from jax import lax
from jax.experimental import pallas as pl
from jax.experimental.pallas import tpu as pltpu
from jax.experimental.pallas import tpu_sc as plsc

N = 10000
DH = 128
E = 320000

NSUB = 16
NCORE = 2
K = 80
E_PER_TILE = 20480
E_PAD = NSUB * E_PER_TILE
CHUNKS = E_PER_TILE // K
N_PAD = 10240
ZROWS = N_PAD // NSUB

BLK = 1000


def _pre_body(h_ref, fd_ref, fs_ref, wr_ref, ur_ref, br_ref, wz_ref, bz_ref,
              w_ref, b_ref, tbl_ref, a_ref, aw_ref):
    h = h_ref[...]
    r = jax.nn.sigmoid(
        jnp.dot(fd_ref[...], wr_ref[...], preferred_element_type=jnp.float32)
        + jnp.dot(h, ur_ref[...], preferred_element_type=jnp.float32)
        + br_ref[...])
    tbl_ref[0] = h
    tbl_ref[1] = r * h
    a_ref[...] = jnp.dot(fs_ref[...], wz_ref[...],
                         preferred_element_type=jnp.float32) + bz_ref[...]
    aw_ref[...] = jnp.dot(fs_ref[...], w_ref[...],
                          preferred_element_type=jnp.float32) + b_ref[...]


def _post_body(s_ref, t_ref, a_ref, aw_ref, uz_ref, u_ref, out_ref):
    s = s_ref[0]
    t = t_ref[0]
    z = jax.nn.sigmoid(
        a_ref[...] + jnp.dot(s, uz_ref[...], preferred_element_type=jnp.float32))
    ht = jnp.tanh(
        aw_ref[...] + jnp.dot(t, u_ref[...], preferred_element_type=jnp.float32))
    out_ref[...] = (1.0 - z) * s + z * ht


_row_spec = pl.BlockSpec((BLK, DH), lambda i: (i, 0))
_w_spec = pl.BlockSpec((DH, DH), lambda i: (0, 0))
_b_spec = pl.BlockSpec((1, DH), lambda i: (0, 0))


def _pre_call(h, f_dst, f_src, wr, ur, br, wz, bz, w, b):
    return pl.pallas_call(
        _pre_body,
        grid=(N // BLK,),
        in_specs=[_row_spec, _row_spec, _row_spec, _w_spec, _w_spec, _b_spec,
                  _w_spec, _b_spec, _w_spec, _b_spec],
        out_specs=[pl.BlockSpec((2, BLK, DH), lambda i: (0, i, 0)),
                   _row_spec, _row_spec],
        out_shape=[jax.ShapeDtypeStruct((2, N, DH), jnp.float32),
                   jax.ShapeDtypeStruct((N, DH), jnp.float32),
                   jax.ShapeDtypeStruct((N, DH), jnp.float32)],
    )(h, f_dst, f_src, wr, ur, br, wz, bz, w, b)


def _post_call(seg, a, aw, uz, u):
    return pl.pallas_call(
        _post_body,
        grid=(N // BLK,),
        in_specs=[pl.BlockSpec((1, BLK, DH), lambda i: (0, i, 0)),
                  pl.BlockSpec((1, BLK, DH), lambda i: (1, i, 0)),
                  _row_spec, _row_spec, _w_spec, _w_spec],
        out_specs=_row_spec,
        out_shape=jax.ShapeDtypeStruct((N, DH), jnp.float32),
    )(seg, seg, a, aw, uz, u)


@functools.partial(
    pl.kernel,
    out_type=jax.ShapeDtypeStruct((NCORE, N_PAD, DH), jnp.float32),
    mesh=plsc.VectorSubcoreMesh(core_axis_name="c", subcore_axis_name="s"),
    scratch_types=[
        pltpu.VMEM((4, 2, K), jnp.int32),
        pltpu.VMEM((4, K, DH), jnp.float32),
        pltpu.VMEM_SHARED((N_PAD, DH), jnp.float32),
        pltpu.SemaphoreType.DMA,
        pltpu.SemaphoreType.DMA,
        pltpu.SemaphoreType.DMA,
        pltpu.SemaphoreType.DMA,
        pltpu.SemaphoreType.DMA,
        pltpu.SemaphoreType.DMA,
        pltpu.SemaphoreType.DMA,
        pltpu.SemaphoreType.DMA,
        pltpu.SemaphoreType.DMA,
        pltpu.SemaphoreType.DMA,
        pltpu.SemaphoreType.DMA,
        pltpu.SemaphoreType.DMA,
    ],
)
def _segsum_sc(tbl_hbm, ecomb_hbm, out_hbm, ibuf, rows, accum, *sems):
    semA = sems[0:4]
    semG = sems[4:8]
    semS = sems[8:12]
    c = lax.axis_index("c")
    sid = lax.axis_index("s")

    zero16 = jnp.zeros((16,), jnp.float32)

    def zrow(r, carry):
        for kk in range(DH // 16):
            rows[0, r, pl.ds(kk * 16, 16)] = zero16
        return carry

    lax.fori_loop(0, K, zrow, 0)
    zbase = sid * ZROWS
    for j in range(ZROWS // K):
        pltpu.sync_copy(rows.at[0], accum.at[pl.ds(zbase + j * K, K)])
    plsc.subcore_barrier()

    cbase = sid * CHUNKS

    def idxload(i, q):
        pltpu.async_copy(ecomb_hbm.at[c, cbase + i], ibuf.at[q], semA[q])

    def idxwait(i, q):
        pltpu.make_async_copy(ecomb_hbm.at[c, cbase + i], ibuf.at[q],
                              semA[q]).wait()

    H = K // 2

    def gather(i, q):
        pltpu.async_copy(tbl_hbm.at[ibuf.at[q, 0, pl.ds(0, H)]],
                         rows.at[q, pl.ds(0, H)], semG[q])
        pltpu.async_copy(tbl_hbm.at[ibuf.at[q, 0, pl.ds(H, H)]],
                         rows.at[q, pl.ds(H, H)], semG[q])

    def gatherwait(q):
        pltpu.make_async_copy(tbl_hbm.at[ibuf.at[q, 0, pl.ds(0, H)]],
                              rows.at[q, pl.ds(0, H)], semG[q]).wait()
        pltpu.make_async_copy(tbl_hbm.at[ibuf.at[q, 0, pl.ds(H, H)]],
                              rows.at[q, pl.ds(H, H)], semG[q]).wait()

    def scatterstart(q):
        pltpu.async_copy(rows.at[q], accum.at[ibuf.at[q, 1]], semS[q],
                         add=True)

    def scatterwait(q):
        pltpu.make_async_copy(rows.at[q], accum.at[ibuf.at[q, 1]],
                              semS[q]).wait()

    def body(i, q):
        q2 = (q + 2) % 4
        scatterwait(q2)
        idxload(i + 2, q2)
        gatherwait(q)
        scatterstart(q)
        idxwait(i + 2, q2)
        gather(i + 2, q2)

    for q in range(4):
        idxload(q, q)
    idxwait(0, 0)
    gather(0, 0)
    idxwait(1, 1)
    gather(1, 1)
    gatherwait(0)
    scatterstart(0)
    idxwait(2, 2)
    gather(2, 2)
    gatherwait(1)
    scatterstart(1)
    idxwait(3, 3)
    gather(3, 3)

    def pipe(g, carry):
        i = g * 4 + 2
        body(i, 2)
        body(i + 1, 3)
        body(i + 2, 0)
        body(i + 3, 1)
        return carry

    lax.fori_loop(0, (CHUNKS - 4) // 4, pipe, 0)
    scatterwait(0)
    gatherwait(2)
    scatterstart(2)
    scatterwait(1)
    gatherwait(3)
    scatterstart(3)
    scatterwait(2)
    scatterwait(3)
    plsc.subcore_barrier()

    for j in range(ZROWS // K):
        pltpu.sync_copy(accum.at[pl.ds(zbase + j * K, K)], rows.at[0])
        pltpu.sync_copy(rows.at[0], out_hbm.at[c, pl.ds(zbase + j * K, K)])


def kernel(h, f_src, f_dst, edge_index, wz, uz, bz, wr, ur, br, w, u, b):
    src = edge_index[0]
    dst = edge_index[1]
    pad = E_PAD - E
    src_p = jnp.concatenate([src, jnp.zeros((pad,), jnp.int32)])
    dst_p = jnp.concatenate([dst, jnp.full((pad,), N, jnp.int32)])
    plane0 = jnp.stack([src_p.reshape(-1, K), dst_p.reshape(-1, K)], axis=1)
    off = jnp.array([N, 0], jnp.int32).reshape(1, 2, 1)
    ecomb = jnp.stack([plane0, plane0 + off])
    tbl, a, aw = _pre_call(h, f_dst, f_src, wr, ur, br, wz, bz, w, b)
    seg = _segsum_sc(tbl.reshape(2 * N, DH), ecomb)
    return _post_call(seg, a, aw, uz, u)
